# baseline (speedup 1.0000x reference)
import jax
import jax.numpy as jnp
from jax.experimental import pallas as pl
from jax.experimental.pallas import tpu as pltpu

_CELLS = 2


def _encoder_body(x_ref, bn_scale_ref, bn_shift_ref, w0_ref, b0_ref,
                  w1_ref, b1_ref, o_ref, pad_ref, patch_ref):
    H = x_ref.shape[1]
    W = x_ref.shape[2]
    C = x_ref.shape[3]
    HW = H * W

    pad_ref[...] = jnp.zeros(pad_ref.shape, pad_ref.dtype)

    def bn_conv(x2d, bn_row, w, b):
        scale = bn_scale_ref[bn_row:bn_row + 1, :]
        shift = bn_shift_ref[bn_row:bn_row + 1, :]
        bnx = (x2d * scale + shift).astype(jnp.bfloat16)
        pad_ref[1:H + 1, 8:8 + W, :] = bnx.reshape(H, W, C)
        for kh in range(3):
            for kw in range(3):
                t = kh * 3 + kw
                patch_ref[:, t * C:(t + 1) * C] = (
                    pad_ref[kh:kh + H, 7 + kw:7 + kw + W, :].reshape(HW, C))
        return jnp.dot(patch_ref[...], w,
                       preferred_element_type=jnp.float32) + b

    cell_in = x_ref[0].reshape(HW, C).astype(jnp.float32)
    for c in range(_CELLS):
        y0 = bn_conv(cell_in, 2 * c + 0, w0_ref[c], b0_ref[c])
        node1 = jnp.maximum(y0[:, :C], 0.0)
        y1 = bn_conv(node1, 2 * c + 1, w1_ref[c], b1_ref[c])
        cell_in = y0[:, C:] + jnp.maximum(y1, 0.0)

    o_ref[0] = jnp.sum(cell_in, axis=0, keepdims=True) * (1.0 / HW)


def _head_body(p_ref, hw_ref, hb_ref, o_ref):
    o_ref[...] = jnp.dot(p_ref[...], hw_ref[...],
                         preferred_element_type=jnp.float32) + hb_ref[...]


def kernel(x, bn_scale, bn_shift, w0, b0, w1, b1, head_w, head_b):
    x = jnp.transpose(x, (0, 2, 3, 1)).astype(jnp.bfloat16)
    B, H, W, C = x.shape
    K = head_w.shape[1]
    nine_c = 9 * C

    pooled = pl.pallas_call(
        _encoder_body,
        out_shape=jax.ShapeDtypeStruct((B, 1, C), jnp.float32),
        grid=(B,),
        in_specs=[
            pl.BlockSpec((1, H, W, C), lambda b: (b, 0, 0, 0)),
            pl.BlockSpec((2 * _CELLS, C), lambda b: (0, 0)),
            pl.BlockSpec((2 * _CELLS, C), lambda b: (0, 0)),
            pl.BlockSpec((_CELLS, nine_c, 2 * C), lambda b: (0, 0, 0)),
            pl.BlockSpec((_CELLS, 1, 2 * C), lambda b: (0, 0, 0)),
            pl.BlockSpec((_CELLS, nine_c, C), lambda b: (0, 0, 0)),
            pl.BlockSpec((_CELLS, 1, C), lambda b: (0, 0, 0)),
        ],
        out_specs=pl.BlockSpec((1, 1, C), lambda b: (b, 0, 0)),
        scratch_shapes=[
            pltpu.VMEM((H + 2, W + 16, C), jnp.bfloat16),
            pltpu.VMEM((H * W, nine_c), jnp.bfloat16),
        ],
        compiler_params=pltpu.CompilerParams(dimension_semantics=("parallel",)),
    )(x, bn_scale, bn_shift, w0.astype(jnp.bfloat16), b0,
      w1.astype(jnp.bfloat16), b1)

    logits = pl.pallas_call(
        _head_body,
        out_shape=jax.ShapeDtypeStruct((B, K), jnp.float32),
    )(pooled.reshape(B, C), head_w, head_b)
    return logits
